# Initial kernel scaffold; baseline (speedup 1.0000x reference)
import functools

import jax
import jax.numpy as jnp
from jax import lax
from jax.experimental import pallas as pl
from jax.experimental.pallas import tpu as pltpu
from jax.experimental.pallas import tpu_sc as plsc

NC = 2
NS = 16
CH = 128


def _sc_mesh():
    return plsc.VectorSubcoreMesh(
        core_axis_name="c", subcore_axis_name="s", num_cores=NC, num_subcores=NS
    )


def _zero_rows(buf, nrows, ncols):
    z = jnp.zeros((16,), jnp.float32)

    def body(i, _):
        for k in range(ncols // 16):
            buf[i, pl.ds(k * 16, 16)] = z
        return 0

    lax.fori_loop(0, nrows, body, 0)


def _make_deg(e_pad, acc_n):
    nch = e_pad // (NC * NS * CH)
    zper = acc_n // (NC * NS)
    oper = acc_n // NS

    def body(dst_hbm, out_hbm, acc, idx_v, ones_v, zero_v):
        c = lax.axis_index("c")
        s = lax.axis_index("s")
        wid = s * NC + c

        _zero_rows(zero_v, 1, zper)
        o = jnp.ones((16,), jnp.float32)
        for k in range(CH // 16):
            ones_v[pl.ds(k * 16, 16)] = o
        pltpu.sync_copy(zero_v.at[0], acc.at[pl.ds(wid * zper, zper)])
        plsc.subcore_barrier()

        pltpu.sync_copy(dst_hbm.at[pl.ds(wid * nch, nch)], idx_v)

        def chunk(j, _):
            pltpu.sync_copy(ones_v, acc.at[idx_v.at[j]], add=True)
            return 0

        lax.fori_loop(0, nch, chunk, 0)
        plsc.subcore_barrier()
        pltpu.sync_copy(
            acc.at[pl.ds(s * oper, oper)],
            out_hbm.at[pl.ds(c * acc_n + s * oper, oper)],
        )

    return pl.kernel(
        body,
        out_type=jax.ShapeDtypeStruct((NC * acc_n,), jnp.float32),
        mesh=_sc_mesh(),
        scratch_types=[
            pltpu.VMEM_SHARED((acc_n,), jnp.float32),
            pltpu.VMEM((nch, CH), jnp.int32),
            pltpu.VMEM((CH,), jnp.float32),
            pltpu.VMEM((1, zper), jnp.float32),
        ],
    )


def _make_agg(n, f, e_pad, acc_n, feature_split):
    workers = NS if feature_split else NC * NS
    nch = e_pad // (workers * CH)
    assert nch % 2 == 0
    zrep = acc_n // (NS * CH)
    oper = n // NS

    def body(src_hbm, dst_hbm, tab_hbm, out_hbm,
             acc, sidx, didx, rows0, rows1, sem0, sem1):
        c = lax.axis_index("c")
        s = lax.axis_index("s")
        base = (s if feature_split else s * NC + c) * nch

        _zero_rows(rows0, CH, f)
        for k in range(zrep):
            pltpu.sync_copy(rows0, acc.at[pl.ds(s * zrep * CH + k * CH, CH)])
        plsc.subcore_barrier()

        pltpu.sync_copy(src_hbm.at[pl.ds(base, nch)], sidx)
        pltpu.sync_copy(dst_hbm.at[pl.ds(base, nch)], didx)

        if feature_split:
            off = c * n

            def shift(i, _):
                for k in range(CH // 16):
                    sl = pl.ds(k * 16, 16)
                    sidx[i, sl] = sidx[i, sl] + off
                return 0

            lax.fori_loop(0, nch, shift, 0)

        pltpu.async_copy(tab_hbm.at[sidx.at[0]], rows0, sem0)
        pltpu.async_copy(tab_hbm.at[sidx.at[1]], rows1, sem1)

        def pair(g, _):
            j = 2 * g
            pltpu.make_async_copy(tab_hbm.at[sidx.at[j]], rows0, sem0).wait()
            pltpu.sync_copy(rows0, acc.at[didx.at[j]], add=True)
            pltpu.async_copy(tab_hbm.at[sidx.at[j + 2]], rows0, sem0)
            pltpu.make_async_copy(tab_hbm.at[sidx.at[j + 1]], rows1, sem1).wait()
            pltpu.sync_copy(rows1, acc.at[didx.at[j + 1]], add=True)
            pltpu.async_copy(tab_hbm.at[sidx.at[j + 3]], rows1, sem1)
            return 0

        lax.fori_loop(0, nch // 2 - 1, pair, 0)
        j = nch - 2
        pltpu.make_async_copy(tab_hbm.at[sidx.at[j]], rows0, sem0).wait()
        pltpu.sync_copy(rows0, acc.at[didx.at[j]], add=True)
        pltpu.make_async_copy(tab_hbm.at[sidx.at[j + 1]], rows1, sem1).wait()
        pltpu.sync_copy(rows1, acc.at[didx.at[j + 1]], add=True)

        plsc.subcore_barrier()
        pltpu.sync_copy(
            acc.at[pl.ds(s * oper, oper)],
            out_hbm.at[pl.ds(c * n + s * oper, oper)],
        )

    return pl.kernel(
        body,
        out_type=jax.ShapeDtypeStruct((NC * n, f), jnp.float32),
        mesh=_sc_mesh(),
        scratch_types=[
            pltpu.VMEM_SHARED((acc_n, f), jnp.float32),
            pltpu.VMEM((nch, CH), jnp.int32),
            pltpu.VMEM((nch, CH), jnp.int32),
            pltpu.VMEM((CH, f), jnp.float32),
            pltpu.VMEM((CH, f), jnp.float32),
            pltpu.SemaphoreType.DMA,
            pltpu.SemaphoreType.DMA,
        ],
    )


def _dinv_from(degt_ref):
    deg = degt_ref[0, 0, :] + degt_ref[0, 1, :] + 1.0
    return lax.rsqrt(deg)


def _mm1_body(x_ref, w_ref, degt_ref, o_ref):
    dinv = _dinv_from(degt_ref)
    h = jnp.dot(x_ref[...], w_ref[...], preferred_element_type=jnp.float32)
    o_ref[0] = h * dinv[:, None]


def _mid_body(agg_ref, hs_ref, degt_ref, b1_ref, w2_ref, o_ref):
    dinv = _dinv_from(degt_ref)
    f1 = agg_ref.shape[2]
    r0 = jnp.maximum(
        dinv[:, None] * (agg_ref[0] + hs_ref[0]) + b1_ref[0, 0:1, :], 0.0)
    r1 = jnp.maximum(
        dinv[:, None] * (agg_ref[1] + hs_ref[1]) + b1_ref[1, 0:1, :], 0.0)
    h2 = jnp.dot(r0, w2_ref[0:f1, :], preferred_element_type=jnp.float32)
    h2 = h2 + jnp.dot(r1, w2_ref[f1:2 * f1, :],
                      preferred_element_type=jnp.float32)
    o_ref[...] = h2 * dinv[:, None]


def _fin_body(agg_ref, hs_ref, degt_ref, b2_ref, o_ref):
    dinv = _dinv_from(degt_ref)
    o_ref[...] = (dinv[:, None] * (agg_ref[0] + agg_ref[1] + hs_ref[...])
                  + b2_ref[0:1, :])


@jax.jit
def kernel(x, edge_index, W1, b1, W2, b2):
    n, d_in = x.shape
    dh = W1.shape[1]
    dc = W2.shape[1]
    e = edge_index.shape[1]
    f1 = dh // NC

    grain = NC * NS * CH
    e_pad = ((e + grain - 1) // grain) * grain
    acc_n = ((n + NS * CH - 1) // (NS * CH)) * (NS * CH)
    r = n // 10

    ei = edge_index.astype(jnp.int32)
    pad = e_pad - e
    pad_ids = jnp.arange(pad, dtype=jnp.int32)
    src_p = jnp.concatenate([ei[0], pad_ids % jnp.int32(n)])
    dst_p = jnp.concatenate([ei[1], jnp.int32(n) + pad_ids % jnp.int32(acc_n - n)])
    src2d = src_p.reshape(e_pad // CH, CH)
    dst2d = dst_p.reshape(e_pad // CH, CH)

    degf = _make_deg(e_pad, acc_n)(dst2d)
    degt = (degf.reshape(NC, acc_n)[:, :n]
            .reshape(NC, n // r, r).transpose(1, 0, 2))

    hs1 = pl.pallas_call(
        _mm1_body,
        grid=(n // r, NC),
        in_specs=[
            pl.BlockSpec((r, d_in), lambda i, c: (i, 0)),
            pl.BlockSpec((d_in, f1), lambda i, c: (0, c)),
            pl.BlockSpec((1, NC, r), lambda i, c: (i, 0, 0)),
        ],
        out_specs=pl.BlockSpec((1, r, f1), lambda i, c: (c, i, 0)),
        out_shape=jax.ShapeDtypeStruct((NC, n, f1), jnp.float32),
    )(x, W1, degt)

    agg1 = _make_agg(n, f1, e_pad, acc_n, feature_split=True)(
        src2d, dst2d, hs1.reshape(NC * n, f1)
    ).reshape(NC, n, f1)

    b1b = jnp.broadcast_to(b1.reshape(NC, 1, f1), (NC, 8, f1))
    hs2 = pl.pallas_call(
        _mid_body,
        grid=(n // r,),
        in_specs=[
            pl.BlockSpec((NC, r, f1), lambda i: (0, i, 0)),
            pl.BlockSpec((NC, r, f1), lambda i: (0, i, 0)),
            pl.BlockSpec((1, NC, r), lambda i: (i, 0, 0)),
            pl.BlockSpec((NC, 8, f1), lambda i: (0, 0, 0)),
            pl.BlockSpec((dh, dc), lambda i: (0, 0)),
        ],
        out_specs=pl.BlockSpec((r, dc), lambda i: (i, 0)),
        out_shape=jax.ShapeDtypeStruct((n, dc), jnp.float32),
    )(agg1, hs1, degt, b1b, W2)

    agg2 = _make_agg(n, dc, e_pad, acc_n, feature_split=False)(
        src2d, dst2d, hs2
    ).reshape(NC, n, dc)

    b2b = jnp.broadcast_to(b2.reshape(1, dc), (8, dc))
    out = pl.pallas_call(
        _fin_body,
        grid=(n // r,),
        in_specs=[
            pl.BlockSpec((NC, r, dc), lambda i: (0, i, 0)),
            pl.BlockSpec((r, dc), lambda i: (i, 0)),
            pl.BlockSpec((1, NC, r), lambda i: (i, 0, 0)),
            pl.BlockSpec((8, dc), lambda i: (0, 0)),
        ],
        out_specs=pl.BlockSpec((r, dc), lambda i: (i, 0)),
        out_shape=jax.ShapeDtypeStruct((n, dc), jnp.float32),
    )(agg2, hs2, degt, b2b)

    return out


# speedup vs baseline: 18.0650x; 18.0650x over previous
import functools

import jax
import jax.numpy as jnp
from jax import lax
from jax.experimental import pallas as pl
from jax.experimental.pallas import tpu as pltpu
from jax.experimental.pallas import tpu_sc as plsc

NC = 2
NS = 16
CH = 64


def _sc_mesh():
    return plsc.VectorSubcoreMesh(
        core_axis_name="c", subcore_axis_name="s", num_cores=NC, num_subcores=NS
    )


def _zero_rows(buf, nrows, ncols):
    z = jnp.zeros((16,), jnp.float32)

    def body(i, _):
        for k in range(ncols // 16):
            buf[i, pl.ds(k * 16, 16)] = z
        return 0

    lax.fori_loop(0, nrows, body, 0)


def _make_deg(e_pad, acc_n):
    nch = e_pad // (NC * NS * CH)
    zper = acc_n // (NC * NS)
    oper = acc_n // NS

    def body(dst_hbm, out_hbm, acc, idx_v, ones_v, zero_v):
        c = lax.axis_index("c")
        s = lax.axis_index("s")
        wid = s * NC + c

        _zero_rows(zero_v, 1, zper)
        o = jnp.ones((16,), jnp.float32)
        for k in range(CH // 16):
            ones_v[pl.ds(k * 16, 16)] = o
        pltpu.sync_copy(zero_v.at[0], acc.at[pl.ds(wid * zper, zper)])
        plsc.subcore_barrier()

        pltpu.sync_copy(dst_hbm.at[pl.ds(wid * nch, nch)], idx_v)

        def chunk(j, _):
            pltpu.sync_copy(ones_v, acc.at[idx_v.at[j]], add=True)
            return 0

        lax.fori_loop(0, nch, chunk, 0)
        plsc.subcore_barrier()
        pltpu.sync_copy(
            acc.at[pl.ds(s * oper, oper)],
            out_hbm.at[pl.ds(c * acc_n + s * oper, oper)],
        )

    return pl.kernel(
        body,
        out_type=jax.ShapeDtypeStruct((NC * acc_n,), jnp.float32),
        mesh=_sc_mesh(),
        scratch_types=[
            pltpu.VMEM_SHARED((acc_n,), jnp.float32),
            pltpu.VMEM((nch, CH), jnp.int32),
            pltpu.VMEM((CH,), jnp.float32),
            pltpu.VMEM((1, zper), jnp.float32),
        ],
    )


WIN = 40


def _make_agg(n, f, e_pad, acc_n, feature_split):
    workers = NS if feature_split else NC * NS
    nch = e_pad // (workers * CH)
    assert nch % WIN == 0 and WIN % 8 == 0
    zrep = acc_n // (NS * CH)
    oper = (-(-n // NS) + 7) // 8 * 8

    def body(src_hbm, dst_hbm, tab_hbm, out_hbm,
             acc, sidx, didx, rows0, rows1, sem0, sem1):
        c = lax.axis_index("c")
        s = lax.axis_index("s")
        base = (s if feature_split else s * NC + c) * nch

        _zero_rows(rows0, CH, f)
        for k in range(zrep):
            pltpu.sync_copy(rows0, acc.at[pl.ds(s * zrep * CH + k * CH, CH)])
        plsc.subcore_barrier()

        def window(w, _):
            wb = base + w * WIN
            pltpu.sync_copy(src_hbm.at[pl.ds(wb, WIN)], sidx)
            pltpu.sync_copy(dst_hbm.at[pl.ds(wb, WIN)], didx)

            if feature_split:
                off = c * n

                def shift(i, _):
                    for k in range(CH // 16):
                        sl = pl.ds(k * 16, 16)
                        sidx[i, sl] = sidx[i, sl] + off
                    return 0

                lax.fori_loop(0, WIN, shift, 0)

            pltpu.async_copy(tab_hbm.at[sidx.at[0]], rows0, sem0)
            pltpu.async_copy(tab_hbm.at[sidx.at[1]], rows1, sem1)

            def pair(g, _):
                j = 2 * g
                pltpu.make_async_copy(tab_hbm.at[sidx.at[j]], rows0, sem0).wait()
                pltpu.sync_copy(rows0, acc.at[didx.at[j]], add=True)
                pltpu.async_copy(tab_hbm.at[sidx.at[j + 2]], rows0, sem0)
                pltpu.make_async_copy(tab_hbm.at[sidx.at[j + 1]], rows1, sem1).wait()
                pltpu.sync_copy(rows1, acc.at[didx.at[j + 1]], add=True)
                pltpu.async_copy(tab_hbm.at[sidx.at[j + 3]], rows1, sem1)
                return 0

            lax.fori_loop(0, WIN // 2 - 1, pair, 0)
            j = WIN - 2
            pltpu.make_async_copy(tab_hbm.at[sidx.at[j]], rows0, sem0).wait()
            pltpu.sync_copy(rows0, acc.at[didx.at[j]], add=True)
            pltpu.make_async_copy(tab_hbm.at[sidx.at[j + 1]], rows1, sem1).wait()
            pltpu.sync_copy(rows1, acc.at[didx.at[j + 1]], add=True)
            return 0

        lax.fori_loop(0, nch // WIN, window, 0)

        plsc.subcore_barrier()
        start = pl.multiple_of(jnp.minimum(s * oper, n - oper), 8)
        pltpu.sync_copy(
            acc.at[pl.ds(start, oper)],
            out_hbm.at[pl.ds(c * n + start, oper)],
        )

    return pl.kernel(
        body,
        out_type=jax.ShapeDtypeStruct((NC * n, f), jnp.float32),
        mesh=_sc_mesh(),
        scratch_types=[
            pltpu.VMEM_SHARED((acc_n, f), jnp.float32),
            pltpu.VMEM((WIN, CH), jnp.int32),
            pltpu.VMEM((WIN, CH), jnp.int32),
            pltpu.VMEM((CH, f), jnp.float32),
            pltpu.VMEM((CH, f), jnp.float32),
            pltpu.SemaphoreType.DMA,
            pltpu.SemaphoreType.DMA,
        ],
    )


def _dinv_from(degt_ref):
    deg = degt_ref[0, 0, :] + degt_ref[0, 1, :] + 1.0
    return lax.rsqrt(deg)


def _mm1_body(x_ref, w_ref, degt_ref, o_ref):
    dinv = _dinv_from(degt_ref)
    h = jnp.dot(x_ref[...], w_ref[...], preferred_element_type=jnp.float32)
    o_ref[0] = h * dinv[:, None]


def _mid_body(agg_ref, hs_ref, degt_ref, b1_ref, w2_ref, o_ref):
    dinv = _dinv_from(degt_ref)
    f1 = agg_ref.shape[2]
    r0 = jnp.maximum(
        dinv[:, None] * (agg_ref[0] + hs_ref[0]) + b1_ref[0, 0:1, :], 0.0)
    r1 = jnp.maximum(
        dinv[:, None] * (agg_ref[1] + hs_ref[1]) + b1_ref[1, 0:1, :], 0.0)
    h2 = jnp.dot(r0, w2_ref[0:f1, :], preferred_element_type=jnp.float32)
    h2 = h2 + jnp.dot(r1, w2_ref[f1:2 * f1, :],
                      preferred_element_type=jnp.float32)
    dc = w2_ref.shape[1]
    o_ref[:, 0:dc] = h2 * dinv[:, None]
    o_ref[:, dc:] = jnp.zeros((o_ref.shape[0], o_ref.shape[1] - dc),
                              jnp.float32)


def _fin_body(agg_ref, hs_ref, degt_ref, b2_ref, o_ref):
    dinv = _dinv_from(degt_ref)
    dc = o_ref.shape[1]
    o_ref[...] = (dinv[:, None]
                  * (agg_ref[0, :, 0:dc] + agg_ref[1, :, 0:dc]
                     + hs_ref[:, 0:dc])
                  + b2_ref[0:1, :])


@jax.jit
def kernel(x, edge_index, W1, b1, W2, b2):
    n, d_in = x.shape
    dh = W1.shape[1]
    dc = W2.shape[1]
    e = edge_index.shape[1]
    f1 = dh // NC
    dcp = ((dc + 127) // 128) * 128

    grain = 2 * NC * NS * CH
    e_pad = ((e + grain - 1) // grain) * grain
    acc_n = ((n + NS * CH - 1) // (NS * CH)) * (NS * CH)
    r = n // 10

    ei = edge_index.astype(jnp.int32)
    pad = e_pad - e
    pad_ids = jnp.arange(pad, dtype=jnp.int32)
    src_p = jnp.concatenate([ei[0], pad_ids % jnp.int32(n)])
    dst_p = jnp.concatenate([ei[1], jnp.int32(n) + pad_ids % jnp.int32(acc_n - n)])
    src2d = src_p.reshape(e_pad // CH, CH)
    dst2d = dst_p.reshape(e_pad // CH, CH)

    degf = _make_deg(e_pad, acc_n)(dst2d)
    degt = (degf.reshape(NC, acc_n)[:, :n]
            .reshape(NC, n // r, r).transpose(1, 0, 2))

    hs1 = pl.pallas_call(
        _mm1_body,
        grid=(n // r, NC),
        in_specs=[
            pl.BlockSpec((r, d_in), lambda i, c: (i, 0)),
            pl.BlockSpec((d_in, f1), lambda i, c: (0, c)),
            pl.BlockSpec((1, NC, r), lambda i, c: (i, 0, 0)),
        ],
        out_specs=pl.BlockSpec((1, r, f1), lambda i, c: (c, i, 0)),
        out_shape=jax.ShapeDtypeStruct((NC, n, f1), jnp.float32),
    )(x, W1, degt)

    agg1 = _make_agg(n, f1, e_pad, acc_n, feature_split=True)(
        src2d, dst2d, hs1.reshape(NC * n, f1)
    ).reshape(NC, n, f1)

    b1b = jnp.broadcast_to(b1.reshape(NC, 1, f1), (NC, 8, f1))
    hs2 = pl.pallas_call(
        _mid_body,
        grid=(n // r,),
        in_specs=[
            pl.BlockSpec((NC, r, f1), lambda i: (0, i, 0)),
            pl.BlockSpec((NC, r, f1), lambda i: (0, i, 0)),
            pl.BlockSpec((1, NC, r), lambda i: (i, 0, 0)),
            pl.BlockSpec((NC, 8, f1), lambda i: (0, 0, 0)),
            pl.BlockSpec((dh, dc), lambda i: (0, 0)),
        ],
        out_specs=pl.BlockSpec((r, dcp), lambda i: (i, 0)),
        out_shape=jax.ShapeDtypeStruct((n, dcp), jnp.float32),
    )(agg1, hs1, degt, b1b, W2)

    agg2 = _make_agg(n, dcp, e_pad, acc_n, feature_split=False)(
        src2d, dst2d, hs2
    ).reshape(NC, n, dcp)

    b2b = jnp.broadcast_to(b2.reshape(1, dc), (8, dc))
    out = pl.pallas_call(
        _fin_body,
        grid=(n // r,),
        in_specs=[
            pl.BlockSpec((NC, r, dcp), lambda i: (0, i, 0)),
            pl.BlockSpec((r, dcp), lambda i: (i, 0)),
            pl.BlockSpec((1, NC, r), lambda i: (i, 0, 0)),
            pl.BlockSpec((8, dc), lambda i: (0, 0)),
        ],
        out_specs=pl.BlockSpec((r, dc), lambda i: (i, 0)),
        out_shape=jax.ShapeDtypeStruct((n, dc), jnp.float32),
    )(agg2, hs2, degt, b2b)

    return out


# speedup vs baseline: 19.7762x; 1.0947x over previous
import functools

import jax
import jax.numpy as jnp
from jax import lax
from jax.experimental import pallas as pl
from jax.experimental.pallas import tpu as pltpu
from jax.experimental.pallas import tpu_sc as plsc

NC = 2
NS = 16
CH = 128


def _sc_mesh():
    return plsc.VectorSubcoreMesh(
        core_axis_name="c", subcore_axis_name="s", num_cores=NC, num_subcores=NS
    )


def _zero_rows(buf, nrows, ncols):
    z = jnp.zeros((16,), jnp.float32)

    def body(i, _):
        for k in range(ncols // 16):
            buf[i, pl.ds(k * 16, 16)] = z
        return 0

    lax.fori_loop(0, nrows, body, 0)


def _make_deg(e_pad, acc_n):
    nch = e_pad // (NC * NS * CH)
    zper = acc_n // (NC * NS)
    oper = acc_n // NS

    def body(dst_hbm, out_hbm, acc, idx_v, ones_v, zero_v):
        c = lax.axis_index("c")
        s = lax.axis_index("s")
        wid = s * NC + c

        _zero_rows(zero_v, 1, zper)
        o = jnp.ones((16,), jnp.float32)
        for k in range(CH // 16):
            ones_v[pl.ds(k * 16, 16)] = o
        pltpu.sync_copy(zero_v.at[0], acc.at[pl.ds(wid * zper, zper)])
        plsc.subcore_barrier()

        pltpu.sync_copy(dst_hbm.at[pl.ds(wid * nch, nch)], idx_v)

        def chunk(j, _):
            pltpu.sync_copy(ones_v, acc.at[idx_v.at[j]], add=True)
            return 0

        lax.fori_loop(0, nch, chunk, 0)
        plsc.subcore_barrier()
        pltpu.sync_copy(
            acc.at[pl.ds(s * oper, oper)],
            out_hbm.at[pl.ds(c * acc_n + s * oper, oper)],
        )

    return pl.kernel(
        body,
        out_type=jax.ShapeDtypeStruct((NC * acc_n,), jnp.float32),
        mesh=_sc_mesh(),
        scratch_types=[
            pltpu.VMEM_SHARED((acc_n,), jnp.float32),
            pltpu.VMEM((nch, CH), jnp.int32),
            pltpu.VMEM((CH,), jnp.float32),
            pltpu.VMEM((1, zper), jnp.float32),
        ],
    )


def _make_agg(n, f, e_pad, acc_n, feature_split, win):
    workers = NS if feature_split else NC * NS
    nch = e_pad // (workers * CH)
    assert nch % win == 0 and win % 8 == 0 and win >= 8
    zrep = acc_n // (NS * CH)
    oper = (-(-n // NS) + 7) // 8 * 8

    def body(src_hbm, dst_hbm, tab_hbm, out_hbm,
             acc, sidx, didx, rows0, rows1, sem0, sem1):
        c = lax.axis_index("c")
        s = lax.axis_index("s")
        base = (s if feature_split else s * NC + c) * nch

        _zero_rows(rows0, CH, f)
        for k in range(zrep):
            pltpu.sync_copy(rows0, acc.at[pl.ds(s * zrep * CH + k * CH, CH)])
        plsc.subcore_barrier()

        def window(w, _):
            wb = base + w * win
            pltpu.sync_copy(src_hbm.at[pl.ds(wb, win)], sidx)
            pltpu.sync_copy(dst_hbm.at[pl.ds(wb, win)], didx)

            if feature_split:
                off = c * n

                def shift(i, _):
                    for k in range(CH // 16):
                        sl = pl.ds(k * 16, 16)
                        sidx[i, sl] = sidx[i, sl] + off
                    return 0

                lax.fori_loop(0, win, shift, 0)

            pltpu.async_copy(tab_hbm.at[sidx.at[0]], rows0, sem0)
            pltpu.async_copy(tab_hbm.at[sidx.at[1]], rows1, sem1)

            def pair(g, _):
                j = 2 * g
                pltpu.make_async_copy(tab_hbm.at[sidx.at[j]], rows0, sem0).wait()
                pltpu.sync_copy(rows0, acc.at[didx.at[j]], add=True)
                pltpu.async_copy(tab_hbm.at[sidx.at[j + 2]], rows0, sem0)
                pltpu.make_async_copy(tab_hbm.at[sidx.at[j + 1]], rows1, sem1).wait()
                pltpu.sync_copy(rows1, acc.at[didx.at[j + 1]], add=True)
                pltpu.async_copy(tab_hbm.at[sidx.at[j + 3]], rows1, sem1)
                return 0

            lax.fori_loop(0, win // 2 - 1, pair, 0)
            j = win - 2
            pltpu.make_async_copy(tab_hbm.at[sidx.at[j]], rows0, sem0).wait()
            pltpu.sync_copy(rows0, acc.at[didx.at[j]], add=True)
            pltpu.make_async_copy(tab_hbm.at[sidx.at[j + 1]], rows1, sem1).wait()
            pltpu.sync_copy(rows1, acc.at[didx.at[j + 1]], add=True)
            return 0

        lax.fori_loop(0, nch // win, window, 0)

        plsc.subcore_barrier()
        start = pl.multiple_of(jnp.minimum(s * oper, n - oper), 8)
        pltpu.sync_copy(
            acc.at[pl.ds(start, oper)],
            out_hbm.at[pl.ds(c * n + start, oper)],
        )

    return pl.kernel(
        body,
        out_type=jax.ShapeDtypeStruct((NC * n, f), jnp.float32),
        mesh=_sc_mesh(),
        scratch_types=[
            pltpu.VMEM_SHARED((acc_n, f), jnp.float32),
            pltpu.VMEM((win, CH), jnp.int32),
            pltpu.VMEM((win, CH), jnp.int32),
            pltpu.VMEM((CH, f), jnp.float32),
            pltpu.VMEM((CH, f), jnp.float32),
            pltpu.SemaphoreType.DMA,
            pltpu.SemaphoreType.DMA,
        ],
    )


def _dinv_from(degt_ref):
    deg = degt_ref[0, 0, :] + degt_ref[0, 1, :] + 1.0
    return lax.rsqrt(deg)


def _mm1_body(x_ref, w_ref, degt_ref, o_ref):
    dinv = _dinv_from(degt_ref)
    h = jnp.dot(x_ref[...], w_ref[...], preferred_element_type=jnp.float32)
    o_ref[0] = h * dinv[:, None]


def _mid_body(agg_ref, hs_ref, degt_ref, b1_ref, w2_ref, o_ref):
    dinv = _dinv_from(degt_ref)
    f1 = agg_ref.shape[2]
    r0 = jnp.maximum(
        dinv[:, None] * (agg_ref[0] + hs_ref[0]) + b1_ref[0, 0:1, :], 0.0)
    r1 = jnp.maximum(
        dinv[:, None] * (agg_ref[1] + hs_ref[1]) + b1_ref[1, 0:1, :], 0.0)
    h2 = jnp.dot(r0, w2_ref[0:f1, :], preferred_element_type=jnp.float32)
    h2 = h2 + jnp.dot(r1, w2_ref[f1:2 * f1, :],
                      preferred_element_type=jnp.float32)
    dc = w2_ref.shape[1]
    o_ref[:, 0:dc] = h2 * dinv[:, None]
    o_ref[:, dc:] = jnp.zeros((o_ref.shape[0], o_ref.shape[1] - dc),
                              jnp.float32)


def _fin_body(agg_ref, hs_ref, degt_ref, b2_ref, o_ref):
    dinv = _dinv_from(degt_ref)
    dc = o_ref.shape[1]
    o_ref[...] = (dinv[:, None]
                  * (agg_ref[0, :, 0:dc] + agg_ref[1, :, 0:dc]
                     + hs_ref[:, 0:dc])
                  + b2_ref[0:1, :])


@jax.jit
def kernel(x, edge_index, W1, b1, W2, b2):
    n, d_in = x.shape
    dh = W1.shape[1]
    dc = W2.shape[1]
    e = edge_index.shape[1]
    f1 = dh // NC
    dcp = ((dc + 127) // 128) * 128

    grain = 2 * NC * NS * CH
    e_pad = ((e + grain - 1) // grain) * grain
    acc_n = ((n + NS * CH - 1) // (NS * CH)) * (NS * CH)
    r = n // 10

    ei = edge_index.astype(jnp.int32)
    pad = e_pad - e
    pad_ids = jnp.arange(pad, dtype=jnp.int32)
    src_p = jnp.concatenate([ei[0], pad_ids % jnp.int32(n)])
    dst_p = jnp.concatenate([ei[1], jnp.int32(n) + pad_ids % jnp.int32(acc_n - n)])
    src2d = src_p.reshape(e_pad // CH, CH)
    dst2d = dst_p.reshape(e_pad // CH, CH)

    degf = _make_deg(e_pad, acc_n)(dst2d)
    degt = (degf.reshape(NC, acc_n)[:, :n]
            .reshape(NC, n // r, r).transpose(1, 0, 2))

    hs1 = pl.pallas_call(
        _mm1_body,
        grid=(n // r, NC),
        in_specs=[
            pl.BlockSpec((r, d_in), lambda i, c: (i, 0)),
            pl.BlockSpec((d_in, f1), lambda i, c: (0, c)),
            pl.BlockSpec((1, NC, r), lambda i, c: (i, 0, 0)),
        ],
        out_specs=pl.BlockSpec((1, r, f1), lambda i, c: (c, i, 0)),
        out_shape=jax.ShapeDtypeStruct((NC, n, f1), jnp.float32),
    )(x, W1, degt)

    agg1 = _make_agg(n, f1, e_pad, acc_n, feature_split=True, win=16)(
        src2d, dst2d, hs1.reshape(NC * n, f1)
    ).reshape(NC, n, f1)

    b1b = jnp.broadcast_to(b1.reshape(NC, 1, f1), (NC, 8, f1))
    hs2 = pl.pallas_call(
        _mid_body,
        grid=(n // r,),
        in_specs=[
            pl.BlockSpec((NC, r, f1), lambda i: (0, i, 0)),
            pl.BlockSpec((NC, r, f1), lambda i: (0, i, 0)),
            pl.BlockSpec((1, NC, r), lambda i: (i, 0, 0)),
            pl.BlockSpec((NC, 8, f1), lambda i: (0, 0, 0)),
            pl.BlockSpec((dh, dc), lambda i: (0, 0)),
        ],
        out_specs=pl.BlockSpec((r, dcp), lambda i: (i, 0)),
        out_shape=jax.ShapeDtypeStruct((n, dcp), jnp.float32),
    )(agg1, hs1, degt, b1b, W2)

    agg2 = _make_agg(n, dcp, e_pad, acc_n, feature_split=False, win=8)(
        src2d, dst2d, hs2
    ).reshape(NC, n, dcp)

    b2b = jnp.broadcast_to(b2.reshape(1, dc), (8, dc))
    out = pl.pallas_call(
        _fin_body,
        grid=(n // r,),
        in_specs=[
            pl.BlockSpec((NC, r, dcp), lambda i: (0, i, 0)),
            pl.BlockSpec((r, dcp), lambda i: (i, 0)),
            pl.BlockSpec((1, NC, r), lambda i: (i, 0, 0)),
            pl.BlockSpec((8, dc), lambda i: (0, 0)),
        ],
        out_specs=pl.BlockSpec((r, dc), lambda i: (i, 0)),
        out_shape=jax.ShapeDtypeStruct((n, dc), jnp.float32),
    )(agg2, hs2, degt, b2b)

    return out


# speedup vs baseline: 20.0709x; 1.0149x over previous
import functools

import jax
import jax.numpy as jnp
from jax import lax
from jax.experimental import pallas as pl
from jax.experimental.pallas import tpu as pltpu
from jax.experimental.pallas import tpu_sc as plsc

NC = 2
NS = 16
CH = 128


def _sc_mesh():
    return plsc.VectorSubcoreMesh(
        core_axis_name="c", subcore_axis_name="s", num_cores=NC, num_subcores=NS
    )


def _zero_rows(buf, nrows, ncols):
    z = jnp.zeros((16,), jnp.float32)

    def body(i, _):
        for k in range(ncols // 16):
            buf[i, pl.ds(k * 16, 16)] = z
        return 0

    lax.fori_loop(0, nrows, body, 0)


def _make_deg(e_pad, acc_n):
    nch = e_pad // (NC * NS * CH)
    zper = acc_n // (NC * NS)
    oper = acc_n // NS

    def body(dst_hbm, out_hbm, acc, idx_v, ones_v, zero_v):
        c = lax.axis_index("c")
        s = lax.axis_index("s")
        wid = s * NC + c

        _zero_rows(zero_v, 1, zper)
        o = jnp.ones((16,), jnp.float32)
        for k in range(CH // 16):
            ones_v[pl.ds(k * 16, 16)] = o
        pltpu.sync_copy(zero_v.at[0], acc.at[pl.ds(wid * zper, zper)])
        plsc.subcore_barrier()

        pltpu.sync_copy(dst_hbm.at[pl.ds(wid * nch, nch)], idx_v)

        def chunk(j, _):
            pltpu.sync_copy(ones_v, acc.at[idx_v.at[j]], add=True)
            return 0

        lax.fori_loop(0, nch, chunk, 0)
        plsc.subcore_barrier()
        pltpu.sync_copy(
            acc.at[pl.ds(s * oper, oper)],
            out_hbm.at[pl.ds(c * acc_n + s * oper, oper)],
        )

    return pl.kernel(
        body,
        out_type=jax.ShapeDtypeStruct((NC * acc_n,), jnp.float32),
        mesh=_sc_mesh(),
        scratch_types=[
            pltpu.VMEM_SHARED((acc_n,), jnp.float32),
            pltpu.VMEM((nch, CH), jnp.int32),
            pltpu.VMEM((CH,), jnp.float32),
            pltpu.VMEM((1, zper), jnp.float32),
        ],
    )


def _make_agg(n, f, e_pad, acc_n, feature_split, win, out_w=None):
    out_w = out_w or f
    workers = NS if feature_split else NC * NS
    nch = e_pad // (workers * CH)
    assert nch % win == 0 and win % 8 == 0 and win >= 8
    zrep = acc_n // (NS * CH)
    oper = (-(-n // NS) + 7) // 8 * 8

    def body(src_hbm, dst_hbm, tab_hbm, out_hbm, *sc):
        acc, sidx, didx, rows0, rows1 = sc[:5]
        if out_w != f:
            crows0, crows1, sem0, sem1 = sc[5:9]
        else:
            crows0, crows1 = rows0, rows1
            sem0, sem1 = sc[5:7]
        c = lax.axis_index("c")
        s = lax.axis_index("s")
        base = (s if feature_split else s * NC + c) * nch

        def compacted(rows, crows):
            if out_w == f:
                return rows

            def cp(i, _):
                for k in range(out_w // 16):
                    sl = pl.ds(k * 16, 16)
                    crows[i, sl] = rows[i, sl]
                return 0

            lax.fori_loop(0, CH, cp, 0)
            return crows

        _zero_rows(crows0, CH, out_w)
        for k in range(zrep):
            pltpu.sync_copy(crows0, acc.at[pl.ds(s * zrep * CH + k * CH, CH)])
        plsc.subcore_barrier()

        def window(w, _):
            wb = base + w * win
            pltpu.sync_copy(src_hbm.at[pl.ds(wb, win)], sidx)
            pltpu.sync_copy(dst_hbm.at[pl.ds(wb, win)], didx)

            if feature_split:
                off = c * n

                def shift(i, _):
                    for k in range(CH // 16):
                        sl = pl.ds(k * 16, 16)
                        sidx[i, sl] = sidx[i, sl] + off
                    return 0

                lax.fori_loop(0, win, shift, 0)

            pltpu.async_copy(tab_hbm.at[sidx.at[0]], rows0, sem0)
            pltpu.async_copy(tab_hbm.at[sidx.at[1]], rows1, sem1)

            def step(j, rows, crows, sem, restart):
                pltpu.make_async_copy(tab_hbm.at[sidx.at[j]], rows, sem).wait()
                if out_w != f:
                    cr = compacted(rows, crows)
                    if restart:
                        pltpu.async_copy(tab_hbm.at[sidx.at[j + 2]], rows, sem)
                    pltpu.sync_copy(cr, acc.at[didx.at[j]], add=True)
                else:
                    pltpu.sync_copy(rows, acc.at[didx.at[j]], add=True)
                    if restart:
                        pltpu.async_copy(tab_hbm.at[sidx.at[j + 2]], rows, sem)

            def pair(g, _):
                j = 2 * g
                step(j, rows0, crows0, sem0, True)
                step(j + 1, rows1, crows1, sem1, True)
                return 0

            lax.fori_loop(0, win // 2 - 1, pair, 0)
            step(win - 2, rows0, crows0, sem0, False)
            step(win - 1, rows1, crows1, sem1, False)
            return 0

        lax.fori_loop(0, nch // win, window, 0)

        plsc.subcore_barrier()
        start = pl.multiple_of(jnp.minimum(s * oper, n - oper), 8)
        pltpu.sync_copy(
            acc.at[pl.ds(start, oper)],
            out_hbm.at[pl.ds(c * n + start, oper)],
        )

    scratch = [
        pltpu.VMEM_SHARED((acc_n, out_w), jnp.float32),
        pltpu.VMEM((win, CH), jnp.int32),
        pltpu.VMEM((win, CH), jnp.int32),
        pltpu.VMEM((CH, f), jnp.float32),
        pltpu.VMEM((CH, f), jnp.float32),
    ]
    if out_w != f:
        scratch += [pltpu.VMEM((CH, out_w), jnp.float32),
                    pltpu.VMEM((CH, out_w), jnp.float32)]
    scratch += [pltpu.SemaphoreType.DMA, pltpu.SemaphoreType.DMA]
    return pl.kernel(
        body,
        out_type=jax.ShapeDtypeStruct((NC * n, out_w), jnp.float32),
        mesh=_sc_mesh(),
        scratch_types=scratch,
    )


def _dinv_from(degt_ref):
    deg = degt_ref[0, 0, :] + degt_ref[0, 1, :] + 1.0
    return lax.rsqrt(deg)


def _mm1_body(x_ref, w_ref, degt_ref, o_ref):
    dinv = _dinv_from(degt_ref)
    h = jnp.dot(x_ref[...], w_ref[...], preferred_element_type=jnp.float32)
    o_ref[0] = h * dinv[:, None]


def _mid_body(agg_ref, hs_ref, degt_ref, b1_ref, w2_ref, o_ref):
    dinv = _dinv_from(degt_ref)
    f1 = agg_ref.shape[2]
    r0 = jnp.maximum(
        dinv[:, None] * (agg_ref[0] + hs_ref[0]) + b1_ref[0, 0:1, :], 0.0)
    r1 = jnp.maximum(
        dinv[:, None] * (agg_ref[1] + hs_ref[1]) + b1_ref[1, 0:1, :], 0.0)
    h2 = jnp.dot(r0, w2_ref[0:f1, :], preferred_element_type=jnp.float32)
    h2 = h2 + jnp.dot(r1, w2_ref[f1:2 * f1, :],
                      preferred_element_type=jnp.float32)
    dc = w2_ref.shape[1]
    o_ref[:, 0:dc] = h2 * dinv[:, None]
    o_ref[:, dc:] = jnp.zeros((o_ref.shape[0], o_ref.shape[1] - dc),
                              jnp.float32)


def _fin_body(agg_ref, hs_ref, degt_ref, b2_ref, o_ref):
    dinv = _dinv_from(degt_ref)
    dc = o_ref.shape[1]
    o_ref[...] = (dinv[:, None]
                  * (agg_ref[0, :, 0:dc] + agg_ref[1, :, 0:dc]
                     + hs_ref[:, 0:dc])
                  + b2_ref[0:1, :])


@jax.jit
def kernel(x, edge_index, W1, b1, W2, b2):
    n, d_in = x.shape
    dh = W1.shape[1]
    dc = W2.shape[1]
    e = edge_index.shape[1]
    f1 = dh // NC
    dcp = ((dc + 127) // 128) * 128

    grain = 2 * NC * NS * CH
    e_pad = ((e + grain - 1) // grain) * grain
    acc_n = ((n + NS * CH - 1) // (NS * CH)) * (NS * CH)
    r = n // 10

    ei = edge_index.astype(jnp.int32)
    pad = e_pad - e
    pad_ids = jnp.arange(pad, dtype=jnp.int32)
    src_p = jnp.concatenate([ei[0], pad_ids % jnp.int32(n)])
    dst_p = jnp.concatenate([ei[1], jnp.int32(n) + pad_ids % jnp.int32(acc_n - n)])
    src2d = src_p.reshape(e_pad // CH, CH)
    dst2d = dst_p.reshape(e_pad // CH, CH)

    degf = _make_deg(e_pad, acc_n)(dst2d)
    degt = (degf.reshape(NC, acc_n)[:, :n]
            .reshape(NC, n // r, r).transpose(1, 0, 2))

    hs1 = pl.pallas_call(
        _mm1_body,
        grid=(n // r, NC),
        in_specs=[
            pl.BlockSpec((r, d_in), lambda i, c: (i, 0)),
            pl.BlockSpec((d_in, f1), lambda i, c: (0, c)),
            pl.BlockSpec((1, NC, r), lambda i, c: (i, 0, 0)),
        ],
        out_specs=pl.BlockSpec((1, r, f1), lambda i, c: (c, i, 0)),
        out_shape=jax.ShapeDtypeStruct((NC, n, f1), jnp.float32),
    )(x, W1, degt)

    agg1 = _make_agg(n, f1, e_pad, acc_n, feature_split=True, win=16)(
        src2d, dst2d, hs1.reshape(NC * n, f1)
    ).reshape(NC, n, f1)

    b1b = jnp.broadcast_to(b1.reshape(NC, 1, f1), (NC, 8, f1))
    hs2 = pl.pallas_call(
        _mid_body,
        grid=(n // r,),
        in_specs=[
            pl.BlockSpec((NC, r, f1), lambda i: (0, i, 0)),
            pl.BlockSpec((NC, r, f1), lambda i: (0, i, 0)),
            pl.BlockSpec((1, NC, r), lambda i: (i, 0, 0)),
            pl.BlockSpec((NC, 8, f1), lambda i: (0, 0, 0)),
            pl.BlockSpec((dh, dc), lambda i: (0, 0)),
        ],
        out_specs=pl.BlockSpec((r, dcp), lambda i: (i, 0)),
        out_shape=jax.ShapeDtypeStruct((n, dcp), jnp.float32),
    )(agg1, hs1, degt, b1b, W2)

    agg2 = _make_agg(n, dcp, e_pad, acc_n, feature_split=False, win=8,
                     out_w=dc)(
        src2d, dst2d, hs2
    ).reshape(NC, n, dc)

    b2b = jnp.broadcast_to(b2.reshape(1, dc), (8, dc))
    out = pl.pallas_call(
        _fin_body,
        grid=(n // r,),
        in_specs=[
            pl.BlockSpec((NC, r, dc), lambda i: (0, i, 0)),
            pl.BlockSpec((r, dcp), lambda i: (i, 0)),
            pl.BlockSpec((1, NC, r), lambda i: (i, 0, 0)),
            pl.BlockSpec((8, dc), lambda i: (0, 0)),
        ],
        out_specs=pl.BlockSpec((r, dc), lambda i: (i, 0)),
        out_shape=jax.ShapeDtypeStruct((n, dc), jnp.float32),
    )(agg2, hs2, degt, b2b)

    return out


# speedup vs baseline: 21.8969x; 1.0910x over previous
import functools

import jax
import jax.numpy as jnp
from jax import lax
from jax.experimental import pallas as pl
from jax.experimental.pallas import tpu as pltpu
from jax.experimental.pallas import tpu_sc as plsc

NC = 2
NS = 16
CH = 64


def _sc_mesh():
    return plsc.VectorSubcoreMesh(
        core_axis_name="c", subcore_axis_name="s", num_cores=NC, num_subcores=NS
    )


def _zero_rows(buf, nrows, ncols):
    z = jnp.zeros((16,), jnp.float32)

    def body(i, _):
        for k in range(ncols // 16):
            buf[i, pl.ds(k * 16, 16)] = z
        return 0

    lax.fori_loop(0, nrows, body, 0)


def _make_deg(e_pad, acc_n):
    nch = e_pad // (NC * NS * CH)
    zper = acc_n // (NC * NS)
    oper = acc_n // NS

    def body(dst_hbm, out_hbm, acc, idx_v, ones_v, zero_v):
        c = lax.axis_index("c")
        s = lax.axis_index("s")
        wid = s * NC + c

        _zero_rows(zero_v, 1, zper)
        o = jnp.ones((16,), jnp.float32)
        for k in range(CH // 16):
            ones_v[pl.ds(k * 16, 16)] = o
        pltpu.sync_copy(zero_v.at[0], acc.at[pl.ds(wid * zper, zper)])
        plsc.subcore_barrier()

        pltpu.sync_copy(dst_hbm.at[pl.ds(wid * nch, nch)], idx_v)

        def chunk(j, _):
            pltpu.sync_copy(ones_v, acc.at[idx_v.at[j]], add=True)
            return 0

        lax.fori_loop(0, nch, chunk, 0)
        plsc.subcore_barrier()
        pltpu.sync_copy(
            acc.at[pl.ds(s * oper, oper)],
            out_hbm.at[pl.ds(c * acc_n + s * oper, oper)],
        )

    return pl.kernel(
        body,
        out_type=jax.ShapeDtypeStruct((NC * acc_n,), jnp.float32),
        mesh=_sc_mesh(),
        scratch_types=[
            pltpu.VMEM_SHARED((acc_n,), jnp.float32),
            pltpu.VMEM((nch, CH), jnp.int32),
            pltpu.VMEM((CH,), jnp.float32),
            pltpu.VMEM((1, zper), jnp.float32),
        ],
    )


def _make_agg(n, f, e_pad, acc_n, feature_split, win, out_w=None, nbuf=2):
    out_w = out_w or f
    workers = NS if feature_split else NC * NS
    nch = e_pad // (workers * CH)
    assert nch % win == 0 and win % 8 == 0 and win >= 8 and win % nbuf == 0
    zrep = acc_n // (NS * CH)
    oper = (-(-n // NS) + 7) // 8 * 8

    def body(src_hbm, dst_hbm, tab_hbm, out_hbm, *sc):
        acc, sidx, didx = sc[:3]
        rowsl = list(sc[3:3 + nbuf])
        k = 3 + nbuf
        if out_w != f:
            crowsl = [sc[k], sc[k + 1]]
            k += 2
        else:
            crowsl = [rowsl[0], rowsl[1 % nbuf]]
        seml = list(sc[k:k + nbuf])
        c = lax.axis_index("c")
        s = lax.axis_index("s")
        base = (s if feature_split else s * NC + c) * nch

        def compacted(rows, crows):
            if out_w == f:
                return rows

            def cp(i, _):
                for k in range(out_w // 16):
                    sl = pl.ds(k * 16, 16)
                    crows[i, sl] = rows[i, sl]
                return 0

            lax.fori_loop(0, CH, cp, 0)
            return crows

        _zero_rows(crowsl[0], CH, out_w)
        for k in range(zrep):
            pltpu.sync_copy(crowsl[0],
                            acc.at[pl.ds(s * zrep * CH + k * CH, CH)])
        plsc.subcore_barrier()

        def window(w, _):
            wb = base + w * win
            pltpu.sync_copy(src_hbm.at[pl.ds(wb, win)], sidx)
            pltpu.sync_copy(dst_hbm.at[pl.ds(wb, win)], didx)

            if feature_split:
                off = c * n

                def shift(i, _):
                    for k in range(CH // 16):
                        sl = pl.ds(k * 16, 16)
                        sidx[i, sl] = sidx[i, sl] + off
                    return 0

                lax.fori_loop(0, win, shift, 0)

            def gath(j, b):
                pltpu.async_copy(tab_hbm.at[sidx.at[j]], rowsl[b], seml[b])

            def gwait(b):
                pltpu.make_async_copy(tab_hbm.at[sidx.at[0]], rowsl[b],
                                      seml[b]).wait()

            for b in range(nbuf):
                gath(b, b)

            def step(j, b, restart):
                rows, sem = rowsl[b], seml[b]
                gwait(b)
                if out_w != f:
                    cr = compacted(rows, crowsl[b % 2])
                    if restart:
                        gath(j + nbuf, b)
                    pltpu.sync_copy(cr, acc.at[didx.at[j]], add=True)
                else:
                    pltpu.sync_copy(rows, acc.at[didx.at[j]], add=True)
                    if restart:
                        gath(j + nbuf, b)

            def group(g, _):
                for u in range(nbuf):
                    step(nbuf * g + u, u, True)
                return 0

            lax.fori_loop(0, win // nbuf - 1, group, 0)
            for u in range(nbuf):
                step(win - nbuf + u, u, False)
            return 0

        lax.fori_loop(0, nch // win, window, 0)

        plsc.subcore_barrier()
        start = pl.multiple_of(jnp.minimum(s * oper, n - oper), 8)
        pltpu.sync_copy(
            acc.at[pl.ds(start, oper)],
            out_hbm.at[pl.ds(c * n + start, oper)],
        )

    scratch = [
        pltpu.VMEM_SHARED((acc_n, out_w), jnp.float32),
        pltpu.VMEM((win, CH), jnp.int32),
        pltpu.VMEM((win, CH), jnp.int32),
    ] + [pltpu.VMEM((CH, f), jnp.float32)] * nbuf
    if out_w != f:
        scratch += [pltpu.VMEM((CH, out_w), jnp.float32)] * 2
    scratch += [pltpu.SemaphoreType.DMA] * nbuf
    return pl.kernel(
        body,
        out_type=jax.ShapeDtypeStruct((NC * n, out_w), jnp.float32),
        mesh=_sc_mesh(),
        scratch_types=scratch,
    )


def _dinv_from(degt_ref):
    deg = degt_ref[0, 0, :] + degt_ref[0, 1, :] + 1.0
    return lax.rsqrt(deg)


def _mm1_body(x_ref, w_ref, degt_ref, o_ref):
    dinv = _dinv_from(degt_ref)
    h = jnp.dot(x_ref[...], w_ref[...], preferred_element_type=jnp.float32)
    o_ref[0] = h * dinv[:, None]


def _mid_body(agg_ref, hs_ref, degt_ref, b1_ref, w2_ref, o_ref):
    dinv = _dinv_from(degt_ref)
    f1 = agg_ref.shape[2]
    r0 = jnp.maximum(
        dinv[:, None] * (agg_ref[0] + hs_ref[0]) + b1_ref[0, 0:1, :], 0.0)
    r1 = jnp.maximum(
        dinv[:, None] * (agg_ref[1] + hs_ref[1]) + b1_ref[1, 0:1, :], 0.0)
    h2 = jnp.dot(r0, w2_ref[0:f1, :], preferred_element_type=jnp.float32)
    h2 = h2 + jnp.dot(r1, w2_ref[f1:2 * f1, :],
                      preferred_element_type=jnp.float32)
    dc = w2_ref.shape[1]
    o_ref[:, 0:dc] = h2 * dinv[:, None]
    o_ref[:, dc:] = jnp.zeros((o_ref.shape[0], o_ref.shape[1] - dc),
                              jnp.float32)


def _fin_body(agg_ref, hs_ref, degt_ref, b2_ref, o_ref):
    dinv = _dinv_from(degt_ref)
    dc = o_ref.shape[1]
    o_ref[...] = (dinv[:, None]
                  * (agg_ref[0, :, 0:dc] + agg_ref[1, :, 0:dc]
                     + hs_ref[:, 0:dc])
                  + b2_ref[0:1, :])


@jax.jit
def kernel(x, edge_index, W1, b1, W2, b2):
    n, d_in = x.shape
    dh = W1.shape[1]
    dc = W2.shape[1]
    e = edge_index.shape[1]
    f1 = dh // NC
    dcp = ((dc + 127) // 128) * 128

    grain = 2 * NC * NS * CH
    e_pad = ((e + grain - 1) // grain) * grain
    acc_n = ((n + NS * CH - 1) // (NS * CH)) * (NS * CH)
    r = n // 10

    ei = edge_index.astype(jnp.int32)
    pad = e_pad - e
    pad_ids = jnp.arange(pad, dtype=jnp.int32)
    src_p = jnp.concatenate([ei[0], pad_ids % jnp.int32(n)])
    dst_p = jnp.concatenate([ei[1], jnp.int32(n) + pad_ids % jnp.int32(acc_n - n)])
    src2d = src_p.reshape(e_pad // CH, CH)
    dst2d = dst_p.reshape(e_pad // CH, CH)

    degf = _make_deg(e_pad, acc_n)(dst2d)
    degt = (degf.reshape(NC, acc_n)[:, :n]
            .reshape(NC, n // r, r).transpose(1, 0, 2))

    hs1 = pl.pallas_call(
        _mm1_body,
        grid=(n // r, NC),
        in_specs=[
            pl.BlockSpec((r, d_in), lambda i, c: (i, 0)),
            pl.BlockSpec((d_in, f1), lambda i, c: (0, c)),
            pl.BlockSpec((1, NC, r), lambda i, c: (i, 0, 0)),
        ],
        out_specs=pl.BlockSpec((1, r, f1), lambda i, c: (c, i, 0)),
        out_shape=jax.ShapeDtypeStruct((NC, n, f1), jnp.float32),
    )(x, W1, degt)

    agg1 = _make_agg(n, f1, e_pad, acc_n, feature_split=True, win=40,
                     nbuf=4)(
        src2d, dst2d, hs1.reshape(NC * n, f1)
    ).reshape(NC, n, f1)

    b1b = jnp.broadcast_to(b1.reshape(NC, 1, f1), (NC, 8, f1))
    hs2 = pl.pallas_call(
        _mid_body,
        grid=(n // r,),
        in_specs=[
            pl.BlockSpec((NC, r, f1), lambda i: (0, i, 0)),
            pl.BlockSpec((NC, r, f1), lambda i: (0, i, 0)),
            pl.BlockSpec((1, NC, r), lambda i: (i, 0, 0)),
            pl.BlockSpec((NC, 8, f1), lambda i: (0, 0, 0)),
            pl.BlockSpec((dh, dc), lambda i: (0, 0)),
        ],
        out_specs=pl.BlockSpec((r, dcp), lambda i: (i, 0)),
        out_shape=jax.ShapeDtypeStruct((n, dcp), jnp.float32),
    )(agg1, hs1, degt, b1b, W2)

    agg2 = _make_agg(n, dcp, e_pad, acc_n, feature_split=False, win=40,
                     out_w=dc, nbuf=4)(
        src2d, dst2d, hs2
    ).reshape(NC, n, dc)

    b2b = jnp.broadcast_to(b2.reshape(1, dc), (8, dc))
    out = pl.pallas_call(
        _fin_body,
        grid=(n // r,),
        in_specs=[
            pl.BlockSpec((NC, r, dc), lambda i: (0, i, 0)),
            pl.BlockSpec((r, dcp), lambda i: (i, 0)),
            pl.BlockSpec((1, NC, r), lambda i: (i, 0, 0)),
            pl.BlockSpec((8, dc), lambda i: (0, 0)),
        ],
        out_specs=pl.BlockSpec((r, dc), lambda i: (i, 0)),
        out_shape=jax.ShapeDtypeStruct((n, dc), jnp.float32),
    )(agg2, hs2, degt, b2b)

    return out
